# baseline (speedup 1.0000x reference)
import functools

import jax
import jax.numpy as jnp
from jax import lax
from jax.experimental import pallas as pl
from jax.experimental.pallas import tpu as pltpu
from jax.experimental.pallas import tpu_sc as plsc

B = 16384
NUM_WORKERS = 32
BPW = B // NUM_WORKERS
NG = BPW // 16
L = 16

USER_EMB = 64
FEAT_EMB = 16
OUT_D = USER_EMB + 4 * FEAT_EMB
SMALL_D = 3 * FEAT_EMB
SMALL_OFF = (0, 2, 9)
SMALL_ROWS = (2, 7, 21)
ZIP_OFF = USER_EMB + SMALL_D


@functools.partial(
    pl.kernel,
    out_type=jax.ShapeDtypeStruct((B, OUT_D), jnp.float32),
    mesh=plsc.VectorSubcoreMesh(core_axis_name="c", subcore_axis_name="s"),
    compiler_params=pltpu.CompilerParams(use_tc_tiling_on_sc=False,
                                         needs_layout_passes=False),
    scratch_types=[
        pltpu.VMEM((BPW, 5), jnp.int32),
        pltpu.VMEM((2 * BPW,), jnp.int32),
        pltpu.VMEM((BPW, USER_EMB), jnp.float32),
        pltpu.VMEM((BPW, FEAT_EMB), jnp.float32),
        pltpu.VMEM((BPW, SMALL_D), jnp.float32),
        pltpu.VMEM((sum(SMALL_ROWS), FEAT_EMB), jnp.float32),
        [pltpu.SemaphoreType.DMA for _ in range(3)],
        pltpu.SemaphoreType.DMA,
    ],
)
def _lookup_concat(cat_hbm, user_hbm, gender_hbm, age_hbm, occup_hbm,
                   zip_hbm, out_hbm, slab_v, idx_v, user_v, zip_v,
                   small_v, small_tab_v, gsems, wsem):
    wid = lax.axis_index("s") * 2 + lax.axis_index("c")
    base = wid * BPW
    slab_cp = pltpu.async_copy(
        cat_hbm.at[pl.ds(base, BPW), :], slab_v, gsems[2])
    tab_cps = [
        pltpu.async_copy(
            hbm, small_tab_v.at[pl.ds(SMALL_OFF[t], SMALL_ROWS[t]), :],
            wsem)
        for t, hbm in enumerate((gender_hbm, age_hbm, occup_hbm))
    ]
    lane = lax.iota(jnp.int32, L)
    slab_cp.wait()

    def build_idx(i, carry, f, off):
        rows = lane + i * L
        col = jnp.full((L,), f, jnp.int32)
        idx_v[pl.ds(off + i * L, L)] = plsc.load_gather(slab_v, [rows, col])
        return carry

    lax.fori_loop(0, NG, functools.partial(build_idx, f=0, off=0), 0)
    user_cp = pltpu.async_copy(
        user_hbm.at[idx_v.at[pl.ds(0, BPW)]], user_v, gsems[0])
    lax.fori_loop(0, NG, functools.partial(build_idx, f=4, off=BPW), 0)
    zip_cp = pltpu.async_copy(
        zip_hbm.at[idx_v.at[pl.ds(BPW, BPW)]], zip_v, gsems[1])
    for cp in tab_cps:
        cp.wait()

    for t in range(3):

        def body(i, carry, t=t):
            rows = lane + i * L
            idxg = plsc.load_gather(
                slab_v, [rows, jnp.full((L,), 1 + t, jnp.int32)])
            idxg = idxg + SMALL_OFF[t]
            for j in range(FEAT_EMB):
                vals = plsc.load_gather(
                    small_tab_v, [idxg, jnp.full((L,), j, jnp.int32)])
                col = jnp.full((L,), t * FEAT_EMB + j, jnp.int32)
                plsc.store_scatter(small_v, [rows, col], vals)
            return carry

        lax.fori_loop(0, NG, body, 0)

    writes = [pltpu.async_copy(
        small_v, out_hbm.at[pl.ds(base, BPW), pl.ds(USER_EMB, SMALL_D)],
        wsem)]
    zip_cp.wait()
    writes.append(pltpu.async_copy(
        zip_v, out_hbm.at[pl.ds(base, BPW), pl.ds(ZIP_OFF, FEAT_EMB)],
        wsem))
    user_cp.wait()
    writes.append(pltpu.async_copy(
        user_v, out_hbm.at[pl.ds(base, BPW), pl.ds(0, USER_EMB)], wsem))
    for cp in writes:
        cp.wait()


def kernel(categorical_feats, user_table, gender_table, age_table,
           occup_table, zip_table):
    return _lookup_concat(categorical_feats.astype(jnp.int32), user_table,
                          gender_table, age_table, occup_table, zip_table)


# speedup vs baseline: 1.3804x; 1.3804x over previous
import functools

import jax
import jax.numpy as jnp
from jax import lax
from jax.experimental import pallas as pl
from jax.experimental.pallas import tpu as pltpu
from jax.experimental.pallas import tpu_sc as plsc

B = 16384
NUM_WORKERS = 32
BPW = B // NUM_WORKERS
L = 16

USER_EMB = 64
FEAT_EMB = 16
OUT_D = USER_EMB + 4 * FEAT_EMB
SMALL_D = 3 * FEAT_EMB
SMALL_ROWS = 2 + 7 + 21
ZIP_OFF = USER_EMB + SMALL_D


@functools.partial(
    pl.kernel,
    out_type=jax.ShapeDtypeStruct((B, OUT_D), jnp.float32),
    mesh=plsc.VectorSubcoreMesh(core_axis_name="c", subcore_axis_name="s"),
    compiler_params=pltpu.CompilerParams(use_tc_tiling_on_sc=False,
                                         needs_layout_passes=False,
                                         disable_bounds_checks=True,
                                         disable_semaphore_checks=True,
                                         skip_device_barrier=True),
    scratch_types=[
        pltpu.VMEM((5 * BPW,), jnp.int32),
        pltpu.VMEM((BPW, USER_EMB), jnp.float32),
        pltpu.VMEM((BPW, FEAT_EMB), jnp.float32),
        pltpu.VMEM((BPW, SMALL_D), jnp.float32),
        pltpu.VMEM((SMALL_ROWS * FEAT_EMB,), jnp.float32),
        [pltpu.SemaphoreType.DMA for _ in range(3)],
        pltpu.SemaphoreType.DMA,
    ],
)
def _lookup_concat(idx_hbm, user_hbm, small_hbm, zip_hbm, out_hbm,
                   idx_v, user_v, zip_v, small_v, small_tab_v, gsems, wsem):
    wid = lax.axis_index("s") * 2 + lax.axis_index("c")
    base = wid * BPW
    idx_cp = pltpu.async_copy(
        idx_hbm.at[pl.ds(wid * 5 * BPW, 5 * BPW)], idx_v, gsems[2])
    tab_cp = pltpu.async_copy(small_hbm, small_tab_v, wsem)
    idx_cp.wait()
    user_cp = pltpu.async_copy(
        user_hbm.at[idx_v.at[pl.ds(0, BPW)]], user_v, gsems[0])
    zip_cp = pltpu.async_copy(
        zip_hbm.at[idx_v.at[pl.ds(4 * BPW, BPW)]], zip_v, gsems[1])
    tab_cp.wait()

    lane = lax.iota(jnp.int32, L)

    for t in range(3):

        def body(i, carry, t=t):
            idxg = idx_v[pl.ds((1 + t) * BPW + i * L, L)]
            src = idxg * FEAT_EMB
            dst_rows = lane + i * L
            for j in range(FEAT_EMB):
                vals = plsc.load_gather(small_tab_v, [src + j])
                col = jnp.full((L,), t * FEAT_EMB + j, jnp.int32)
                plsc.store_scatter(small_v, [dst_rows, col], vals)
            return carry

        lax.fori_loop(0, BPW // L, body, 0)

    writes = [pltpu.async_copy(
        small_v, out_hbm.at[pl.ds(base, BPW), pl.ds(USER_EMB, SMALL_D)],
        wsem)]
    zip_cp.wait()
    writes.append(pltpu.async_copy(
        zip_v, out_hbm.at[pl.ds(base, BPW), pl.ds(ZIP_OFF, FEAT_EMB)],
        wsem))
    user_cp.wait()
    writes.append(pltpu.async_copy(
        user_v, out_hbm.at[pl.ds(base, BPW), pl.ds(0, USER_EMB)], wsem))
    for cp in writes:
        cp.wait()


def kernel(categorical_feats, user_table, gender_table, age_table,
           occup_table, zip_table):
    small = jnp.concatenate(
        [gender_table, age_table, occup_table], axis=0).reshape(-1)
    idx = categorical_feats.astype(jnp.int32)
    idx = idx + jnp.array([0, 0, 2, 9, 0], jnp.int32)[None, :]
    idx = idx.reshape(NUM_WORKERS, BPW, 5).transpose(0, 2, 1).reshape(-1)
    return _lookup_concat(idx, user_table, small, zip_table)


# speedup vs baseline: 1.7043x; 1.2347x over previous
import functools

import jax
import jax.numpy as jnp
from jax import lax
from jax.experimental import pallas as pl
from jax.experimental.pallas import tpu as pltpu
from jax.experimental.pallas import tpu_sc as plsc

B = 16384
NUM_WORKERS = 32
BPW = B // NUM_WORKERS
L = 16

USER_EMB = 64
FEAT_EMB = 16
OUT_D = USER_EMB + 4 * FEAT_EMB
SMALL_D = 3 * FEAT_EMB
SMALL_ROWS = 2 + 7 + 21
ZIP_OFF = USER_EMB + SMALL_D


@functools.partial(
    pl.kernel,
    out_type=jax.ShapeDtypeStruct((B, OUT_D), jnp.float32),
    mesh=plsc.VectorSubcoreMesh(core_axis_name="c", subcore_axis_name="s"),
    compiler_params=pltpu.CompilerParams(use_tc_tiling_on_sc=False,
                                         needs_layout_passes=False,
                                         disable_bounds_checks=True,
                                         disable_semaphore_checks=True,
                                         skip_device_barrier=True),
    scratch_types=[
        pltpu.VMEM((5 * BPW,), jnp.int32),
        pltpu.VMEM((BPW, USER_EMB), jnp.float32),
        pltpu.VMEM((BPW, FEAT_EMB), jnp.float32),
        pltpu.VMEM((BPW, SMALL_D), jnp.float32),
        pltpu.VMEM((SMALL_ROWS * FEAT_EMB,), jnp.float32),
        [pltpu.SemaphoreType.DMA for _ in range(3)],
        pltpu.SemaphoreType.DMA,
    ],
)
def _lookup_concat(idx_hbm, user_hbm, small_hbm, zip_hbm, out_hbm,
                   idx_v, user_v, zip_v, small_v, small_tab_v, gsems, wsem):
    wid = lax.axis_index("s") * 2 + lax.axis_index("c")
    base = wid * BPW
    idx_cp = pltpu.async_copy(
        idx_hbm.at[pl.ds(wid * 5 * BPW, 5 * BPW)], idx_v, gsems[2])
    tab_cp = pltpu.async_copy(small_hbm, small_tab_v, wsem)
    idx_cp.wait()
    user_cp = pltpu.async_copy(
        user_hbm.at[idx_v.at[pl.ds(0, BPW)]], user_v, gsems[0])
    zip_cp = pltpu.async_copy(
        zip_hbm.at[idx_v.at[pl.ds(4 * BPW, BPW)]], zip_v, gsems[1])
    tab_cp.wait()

    lane = lax.iota(jnp.int32, L)

    for t in range(3):

        @functools.partial(plsc.parallel_loop, 0, BPW // L, unroll=2)
        def body(i, t=t):
            idxg = idx_v[pl.ds((1 + t) * BPW + i * L, L)]
            src = idxg * FEAT_EMB
            dst_rows = lane + i * L
            for j in range(FEAT_EMB):
                vals = plsc.load_gather(small_tab_v, [src + j])
                col = jnp.full((L,), t * FEAT_EMB + j, jnp.int32)
                plsc.store_scatter(small_v, [dst_rows, col], vals)

    writes = [pltpu.async_copy(
        small_v, out_hbm.at[pl.ds(base, BPW), pl.ds(USER_EMB, SMALL_D)],
        wsem)]
    zip_cp.wait()
    writes.append(pltpu.async_copy(
        zip_v, out_hbm.at[pl.ds(base, BPW), pl.ds(ZIP_OFF, FEAT_EMB)],
        wsem))
    user_cp.wait()
    writes.append(pltpu.async_copy(
        user_v, out_hbm.at[pl.ds(base, BPW), pl.ds(0, USER_EMB)], wsem))
    for cp in writes:
        cp.wait()


def kernel(categorical_feats, user_table, gender_table, age_table,
           occup_table, zip_table):
    small = jnp.concatenate(
        [gender_table, age_table, occup_table], axis=0).reshape(-1)
    idx = categorical_feats.astype(jnp.int32)
    idx = idx + jnp.array([0, 0, 2, 9, 0], jnp.int32)[None, :]
    idx = idx.reshape(NUM_WORKERS, BPW, 5).transpose(0, 2, 1).reshape(-1)
    return _lookup_concat(idx, user_table, small, zip_table)
